# baseline (speedup 1.0000x reference)
import functools

import jax
import jax.numpy as jnp
from jax import lax
from jax.experimental import pallas as pl
from jax.experimental.pallas import tpu as pltpu
from jax.experimental.pallas import tpu_sc as plsc

_B = 16384
_L = 50
_D = 64
_NC = 2
_NS = 16
_NW = _NC * _NS
_BAGS_W = _B // _NW
_CHUNK = 16
_NCHUNK = _BAGS_W // _CHUNK
_ROWS = _CHUNK * _L
_NSTREAM = 8
_RPS = _ROWS // _NSTREAM

_V = 1000000
_TCOLS = 7812
_TPW = _TCOLS // _NW
_TREM = _TCOLS - _TPW * _NW
_TITER = _TPW + 2


def _sc_transpose_table(w_t, w_tail):
    mesh = plsc.VectorSubcoreMesh(core_axis_name="c", subcore_axis_name="s")

    @functools.partial(
        pl.kernel,
        out_type=jax.ShapeDtypeStruct((_V * _D,), jnp.float32),
        mesh=mesh,
        compiler_params=pltpu.CompilerParams(needs_layout_passes=False),
        scratch_types=[
            pltpu.VMEM((2, 64, 128), jnp.float32),
            pltpu.VMEM((8192,), jnp.float32),
            pltpu.VMEM((8192,), jnp.float32),
            pltpu.VMEM((64, 64), jnp.float32),
            pltpu.SemaphoreType.DMA,
            pltpu.SemaphoreType.DMA,
            pltpu.SemaphoreType.DMA,
            pltpu.SemaphoreType.DMA,
        ],
    )
    def tbody(wt_hbm, tail_hbm, out_hbm, slab_v, pair0_v, pair1_v, tail_v,
              isem0, isem1, osem0, osem1):
        isems = (isem0, isem1)
        osems = (osem0, osem1)
        pairs = (pair0_v, pair1_v)
        wid = lax.axis_index("s") * _NC + lax.axis_index("c")
        start = wid * _TPW + jnp.minimum(wid, _TREM)
        limit = start + _TPW + (wid < _TREM).astype(jnp.int32)

        lanes = lax.iota(jnp.int32, 16)
        pat = lax.shift_right_logical(lanes, 1) * 128 + (lanes & 1) * _D

        def issue_in(c, slot):
            pltpu.make_async_copy(
                wt_hbm.at[:, pl.ds(pl.multiple_of(c * 128, 128), 128)],
                slab_v.at[slot],
                isems[slot],
            ).start()

        def wait_in(slot):
            pltpu.make_async_copy(
                wt_hbm.at[:, pl.ds(0, 128)], slab_v.at[slot], isems[slot]
            ).wait()

        def wait_out(slot):
            pltpu.make_async_copy(
                pairs[slot], out_hbm.at[pl.ds(0, 8192)], osems[slot]
            ).wait()

        def transpose_slab(slot):
            pvf = pairs[slot]
            for m in range(8):
                base = pat + (m * 1024)
                for k in range(_D):
                    v = slab_v[slot, k, pl.ds(16 * m, 16)]
                    plsc.store_scatter(pvf, [base + k], v)

        issue_in(start, 0)

        def pair_body(p, carry):
            for b in range(2):
                t = 2 * p + b
                col = start + t

                @pl.when(col + 1 < limit)
                def _():
                    issue_in(col + 1, 1 - b)

                @pl.when(col < limit)
                def _():
                    wait_in(b)

                    @pl.when(t >= 2)
                    def _():
                        wait_out(b)

                    transpose_slab(b)
                    pltpu.make_async_copy(
                        pairs[b],
                        out_hbm.at[
                            pl.ds(pl.multiple_of(col * 8192, 8192), 8192)
                        ],
                        osems[b],
                    ).start()
            return carry

        lax.fori_loop(0, _TITER // 2, pair_body, 0)
        wait_out(0)
        wait_out(1)

        @pl.when(wid == _NW - 1)
        def _():
            pltpu.sync_copy(tail_hbm, tail_v)
            for m in range(4):
                base = pat + (m * 1024)
                for k in range(_D):
                    v = tail_v[k, pl.ds(16 * m, 16)]
                    plsc.store_scatter(pair0_v, [base + k], v)
            pltpu.sync_copy(
                pair0_v.at[pl.ds(0, 4096)],
                out_hbm.at[pl.ds(999936 * _D, 4096)],
            )

    return tbody(w_t, w_tail)


def _sc_embedding_bag(idx4, weight):
    mesh = plsc.VectorSubcoreMesh(core_axis_name="c", subcore_axis_name="s")

    @functools.partial(
        pl.kernel,
        out_type=jax.ShapeDtypeStruct((_B, _D), jnp.float32),
        mesh=mesh,
        compiler_params=pltpu.CompilerParams(use_tc_tiling_on_sc=False),
        scratch_types=[
            pltpu.VMEM((_NCHUNK, _NSTREAM, _RPS), jnp.int32),
            pltpu.VMEM((2, _ROWS, _D), jnp.float32),
            pltpu.VMEM((2, _CHUNK, _D), jnp.float32),
            pltpu.SemaphoreType.DMA,
            pltpu.SemaphoreType.DMA,
            pltpu.SemaphoreType.DMA,
            pltpu.SemaphoreType.DMA,
        ],
    )
    def body(idx_hbm, w_hbm, out_hbm, idx_v, rows_v, out_v,
             gsem0, gsem1, osem0, osem1):
        gsems = (gsem0, gsem1)
        osems = (osem0, osem1)
        wid = lax.axis_index("s") * _NC + lax.axis_index("c")
        bag0 = wid * _BAGS_W

        pltpu.sync_copy(idx_hbm.at[wid], idx_v)

        def issue(g, slot):
            for j in range(_NSTREAM):
                pltpu.make_async_copy(
                    w_hbm.at[idx_v.at[g, j]],
                    rows_v.at[slot, pl.ds(j * _RPS, _RPS), :],
                    gsems[slot],
                ).start()

        def drain_gather(slot):
            pltpu.make_async_copy(
                w_hbm.at[pl.ds(0, _ROWS), :], rows_v.at[slot], gsems[slot]
            ).wait()

        def drain_out(slot):
            pltpu.make_async_copy(
                out_v.at[slot], out_hbm.at[pl.ds(0, _CHUNK), :], osems[slot]
            ).wait()

        def compute(g, slot):
            def bag_body(i, c2):
                r0 = i * _L
                for d in range(_D // 16):
                    sl = pl.ds(d * 16, 16)
                    acc = rows_v[slot, r0, sl]
                    for l in range(1, _L):
                        acc = acc + rows_v[slot, r0 + l, sl]
                    out_v[slot, i, sl] = acc * jnp.float32(1.0 / _L)
                return c2

            lax.fori_loop(0, _CHUNK, bag_body, 0)
            pltpu.make_async_copy(
                out_v.at[slot],
                out_hbm.at[pl.ds(bag0 + g * _CHUNK, _CHUNK), :],
                osems[slot],
            ).start()

        issue(0, 0)

        def pair_body(p, carry):
            for b in range(2):
                g = 2 * p + b

                @pl.when(g + 1 < _NCHUNK)
                def _():
                    issue(g + 1, 1 - b)

                drain_gather(b)

                @pl.when(g >= 2)
                def _():
                    drain_out(b)

                compute(g, b)
            return carry

        lax.fori_loop(0, _NCHUNK // 2, pair_body, 0)
        drain_out(0)
        drain_out(1)

    return body(idx4, weight)


def kernel(input, weight):
    table = _sc_transpose_table(weight.T, weight[_V - _D:, :].T)
    idx4 = input.astype(jnp.int32).reshape(_NW, _NCHUNK, _NSTREAM, _RPS)
    return _sc_embedding_bag(idx4, table.reshape(_V, _D))


# speedup vs baseline: 1.8119x; 1.8119x over previous
import functools

import jax
import jax.numpy as jnp
from jax import lax
from jax.experimental import pallas as pl
from jax.experimental.pallas import tpu as pltpu
from jax.experimental.pallas import tpu_sc as plsc

_B = 16384
_L = 50
_D = 64
_NC = 2
_NS = 16
_NW = _NC * _NS
_BAGS_W = _B // _NW
_CHUNK = 16
_NCHUNK = _BAGS_W // _CHUNK
_ROWS = _CHUNK * _L
_NSTREAM = 8
_RPS = _ROWS // _NSTREAM


def _sc_embedding_bag(idx4, weight):
    mesh = plsc.VectorSubcoreMesh(core_axis_name="c", subcore_axis_name="s")

    @functools.partial(
        pl.kernel,
        out_type=jax.ShapeDtypeStruct((_B, _D), jnp.float32),
        mesh=mesh,
        compiler_params=pltpu.CompilerParams(use_tc_tiling_on_sc=False),
        scratch_types=[
            pltpu.VMEM((_NCHUNK, _NSTREAM, _RPS), jnp.int32),
            pltpu.VMEM((2, _ROWS, _D), jnp.float32),
            pltpu.VMEM((2, _CHUNK, _D), jnp.float32),
            pltpu.SemaphoreType.DMA,
            pltpu.SemaphoreType.DMA,
            pltpu.SemaphoreType.DMA,
            pltpu.SemaphoreType.DMA,
        ],
    )
    def body(idx_hbm, w_hbm, out_hbm, idx_v, rows_v, out_v,
             gsem0, gsem1, osem0, osem1):
        gsems = (gsem0, gsem1)
        osems = (osem0, osem1)
        wid = lax.axis_index("s") * _NC + lax.axis_index("c")
        bag0 = wid * _BAGS_W

        pltpu.sync_copy(idx_hbm.at[wid], idx_v)

        def issue(g, slot):
            for j in range(_NSTREAM):
                pltpu.make_async_copy(
                    w_hbm.at[idx_v.at[g, j]],
                    rows_v.at[slot, pl.ds(j * _RPS, _RPS), :],
                    gsems[slot],
                ).start()

        def drain_gather(slot):
            pltpu.make_async_copy(
                w_hbm.at[pl.ds(0, _ROWS), :], rows_v.at[slot], gsems[slot]
            ).wait()

        def drain_out(slot):
            pltpu.make_async_copy(
                out_v.at[slot], out_hbm.at[pl.ds(0, _CHUNK), :], osems[slot]
            ).wait()

        def compute(g, slot):
            def bag_body(i, c2):
                r0 = i * _L
                for d in range(_D // 16):
                    sl = pl.ds(d * 16, 16)
                    acc = rows_v[slot, r0, sl]
                    for l in range(1, _L):
                        acc = acc + rows_v[slot, r0 + l, sl]
                    out_v[slot, i, sl] = acc * jnp.float32(1.0 / _L)
                return c2

            lax.fori_loop(0, _CHUNK, bag_body, 0)
            pltpu.make_async_copy(
                out_v.at[slot],
                out_hbm.at[pl.ds(bag0 + g * _CHUNK, _CHUNK), :],
                osems[slot],
            ).start()

        issue(0, 0)

        def pair_body(p, carry):
            for b in range(2):
                g = 2 * p + b

                @pl.when(g + 1 < _NCHUNK)
                def _():
                    issue(g + 1, 1 - b)

                drain_gather(b)

                @pl.when(g >= 2)
                def _():
                    drain_out(b)

                compute(g, b)
            return carry

        lax.fori_loop(0, _NCHUNK // 2, pair_body, 0)
        drain_out(0)
        drain_out(1)

    return body(idx4, weight)


def kernel(input, weight):
    idx4 = input.astype(jnp.int32).reshape(_NW, _NCHUNK, _NSTREAM, _RPS)
    return _sc_embedding_bag(idx4, weight)
